# baseline (speedup 1.0000x reference)
import functools

import jax
import jax.numpy as jnp
from jax import lax
from jax.experimental import pallas as pl
from jax.experimental.pallas import tpu as pltpu
from jax.experimental.pallas import tpu_sc as plsc

VOCAB = 1000000
D = 64
BATCH = 4096
HIST = 50
B = BATCH * HIST
NC, NS = 2, 16
NW = NC * NS
BPW = B // NW
CHUNK = 800
NCHUNK = BPW // CHUNK

_mesh = plsc.VectorSubcoreMesh(core_axis_name="c", subcore_axis_name="s")


@functools.partial(
    pl.kernel,
    out_type=jax.ShapeDtypeStruct((NW, NCHUNK, CHUNK, D), jnp.float32),
    mesh=_mesh,
    scratch_types=[
        pltpu.VMEM((CHUNK,), jnp.int32),
        pltpu.VMEM((CHUNK, D), jnp.float32),
        pltpu.SemaphoreType.DMA,
    ],
    compiler_params=pltpu.CompilerParams(use_tc_tiling_on_sc=False),
)
def _gather_kernel(idx_hbm, table_hbm, out_hbm, idx_v, rows_v, sem):
    wid = lax.axis_index("s") * NC + lax.axis_index("c")

    def body(i, carry):
        pltpu.sync_copy(idx_hbm.at[wid, i], idx_v)
        pltpu.async_copy(table_hbm.at[idx_v], rows_v, sem).wait()
        pltpu.sync_copy(rows_v, out_hbm.at[wid, i])
        return carry

    lax.fori_loop(0, NCHUNK, body, 0)


def kernel(input_ids, table):
    flat = input_ids.reshape(NW, NCHUNK, CHUNK).astype(jnp.int32)
    out = _gather_kernel(flat, table)
    return out.reshape(BATCH, HIST, D)


# speedup vs baseline: 1.0094x; 1.0094x over previous
import functools

import jax
import jax.numpy as jnp
from jax import lax
from jax.experimental import pallas as pl
from jax.experimental.pallas import tpu as pltpu
from jax.experimental.pallas import tpu_sc as plsc

VOCAB = 1000000
D = 64
BATCH = 4096
HIST = 50
B = BATCH * HIST
NC, NS = 2, 16
NW = NC * NS
BPW = B // NW
CHUNK = 800
NCHUNK = BPW // CHUNK

_mesh = plsc.VectorSubcoreMesh(core_axis_name="c", subcore_axis_name="s")


@functools.partial(
    pl.kernel,
    out_type=jax.ShapeDtypeStruct((NW, NCHUNK, CHUNK, D), jnp.float32),
    mesh=_mesh,
    scratch_types=[
        pltpu.VMEM((NCHUNK, CHUNK), jnp.int32),
        pltpu.VMEM((CHUNK, D), jnp.float32),
        pltpu.VMEM((CHUNK, D), jnp.float32),
        pltpu.SemaphoreType.DMA,
        pltpu.SemaphoreType.DMA,
        pltpu.SemaphoreType.DMA,
        pltpu.SemaphoreType.DMA,
    ],
    compiler_params=pltpu.CompilerParams(use_tc_tiling_on_sc=False),
)
def _gather_kernel(idx_hbm, table_hbm, out_hbm,
                   idx_v, rows0, rows1, sg0, sg1, sw0, sw1):
    wid = lax.axis_index("s") * NC + lax.axis_index("c")
    pltpu.sync_copy(idx_hbm.at[wid], idx_v)

    rows = (rows0, rows1)
    sg = (sg0, sg1)
    sw = (sw0, sw1)
    for i in range(NCHUNK):
        b = i % 2
        if i >= 2:
            pltpu.make_async_copy(rows[b], out_hbm.at[wid, i - 2], sw[b]).wait()
        pltpu.async_copy(table_hbm.at[idx_v.at[i]], rows[b], sg[b])
        if i >= 1:
            pb = (i - 1) % 2
            pltpu.make_async_copy(table_hbm.at[idx_v.at[i - 1]], rows[pb],
                                  sg[pb]).wait()
            pltpu.async_copy(rows[pb], out_hbm.at[wid, i - 1], sw[pb])
    last = NCHUNK - 1
    lb = last % 2
    pltpu.make_async_copy(rows[(last - 1) % 2], out_hbm.at[wid, last - 1],
                          sw[(last - 1) % 2]).wait()
    pltpu.make_async_copy(table_hbm.at[idx_v.at[last]], rows[lb], sg[lb]).wait()
    pltpu.sync_copy(rows[lb], out_hbm.at[wid, last])


def kernel(input_ids, table):
    flat = input_ids.reshape(NW, NCHUNK, CHUNK).astype(jnp.int32)
    out = _gather_kernel(flat, table)
    return out.reshape(BATCH, HIST, D)
